# baseline (speedup 1.0000x reference)
import functools

import jax
import jax.numpy as jnp
from jax import lax
from jax.experimental import pallas as pl
from jax.experimental.pallas import tpu as pltpu
from jax.experimental.pallas import tpu_sc as plsc

N_HEROES = 100000
EMB_DIM = 128
CONTEXT = 9
HIDDEN = 64

CHUNK = 8192
K_STEPS = -(-N_HEROES // CHUNK)
PADN = K_STEPS * CHUNK


def _sc_gather(ctx_pad, emb_table):
    mesh = plsc.VectorSubcoreMesh(core_axis_name="c", subcore_axis_name="s")

    @functools.partial(
        pl.kernel,
        mesh=mesh,
        out_type=jax.ShapeDtypeStruct((16, EMB_DIM), jnp.float32),
        scratch_types=[
            pltpu.VMEM((16,), jnp.int32),
            pltpu.VMEM((16, EMB_DIM), jnp.float32),
            pltpu.SemaphoreType.DMA,
        ],
    )
    def gather_k(ctx_hbm, table_hbm, out_hbm, idx_v, rows_v, sem):
        cid = lax.axis_index("c")
        sid = lax.axis_index("s")

        @pl.when(jnp.logical_and(cid == 0, sid == 0))
        def _():
            pltpu.sync_copy(ctx_hbm, idx_v)
            pltpu.async_copy(table_hbm.at[idx_v], rows_v, sem).wait()
            pltpu.sync_copy(rows_v, out_hbm)

    return gather_k(ctx_pad, emb_table)


def _mlp_body(emb_ref, w1_ref, b1_ref, w2_ref, b2_ref, out_ref, h_ref):
    i = pl.program_id(0)

    @pl.when(i == 0)
    def _():
        acc = jnp.zeros((1, HIDDEN), jnp.float32)
        for j in range(CONTEXT):
            acc += jnp.dot(
                emb_ref[j : j + 1, :],
                w1_ref[EMB_DIM * j : EMB_DIM * (j + 1), :],
                preferred_element_type=jnp.float32,
            )
        h_ref[...] = jnp.maximum(acc + b1_ref[...], 0.0)

    h = h_ref[...]
    logits = (
        jnp.dot(h, w2_ref[...], preferred_element_type=jnp.float32) + b2_ref[...]
    )
    col = i * CHUNK + lax.broadcasted_iota(jnp.int32, (1, CHUNK), 1)
    logits = jnp.where(col < N_HEROES, logits, -1e30)
    out_ref[0:1, pl.ds(i * CHUNK, CHUNK)] = logits

    @pl.when(i == K_STEPS - 1)
    def _():
        full = out_ref[...]
        m = jnp.max(full)
        s = jnp.sum(jnp.exp(full - m))
        out_ref[...] = full - (m + jnp.log(s))


def _tc_mlp(embeds, W1, b1r, W2, b2r):
    return pl.pallas_call(
        _mlp_body,
        grid=(K_STEPS,),
        in_specs=[
            pl.BlockSpec((16, EMB_DIM), lambda i: (0, 0)),
            pl.BlockSpec((CONTEXT * EMB_DIM, HIDDEN), lambda i: (0, 0)),
            pl.BlockSpec((1, HIDDEN), lambda i: (0, 0)),
            pl.BlockSpec((HIDDEN, CHUNK), lambda i: (0, i)),
            pl.BlockSpec((1, CHUNK), lambda i: (0, i)),
        ],
        out_specs=pl.BlockSpec((1, PADN), lambda i: (0, 0)),
        out_shape=jax.ShapeDtypeStruct((1, PADN), jnp.float32),
        scratch_shapes=[pltpu.VMEM((1, HIDDEN), jnp.float32)],
    )(embeds, W1, b1r, W2, b2r)


def kernel(context, emb_table, W1, b1, W2, b2):
    ctx_pad = jnp.zeros((16,), jnp.int32).at[:CONTEXT].set(context.astype(jnp.int32))
    embeds = _sc_gather(ctx_pad, emb_table)
    out = _tc_mlp(embeds, W1, b1.reshape(1, HIDDEN), W2, b2.reshape(1, N_HEROES))
    return out[:, :N_HEROES]


# speedup vs baseline: 1.0930x; 1.0930x over previous
import functools

import jax
import jax.numpy as jnp
from jax import lax
from jax.experimental import pallas as pl
from jax.experimental.pallas import tpu as pltpu
from jax.experimental.pallas import tpu_sc as plsc

N_HEROES = 100000
EMB_DIM = 128
CONTEXT = 9
HIDDEN = 64

NS = 2
K_STEPS = 8
CHUNK = 6400
ROW = NS * CHUNK
PADN = K_STEPS * ROW


def _sc_gather(ctx_pad, emb_table):
    mesh = plsc.VectorSubcoreMesh(core_axis_name="c", subcore_axis_name="s")

    @functools.partial(
        pl.kernel,
        mesh=mesh,
        out_type=jax.ShapeDtypeStruct((16, EMB_DIM), jnp.float32),
        scratch_types=[
            pltpu.VMEM((16,), jnp.int32),
            pltpu.VMEM((16, EMB_DIM), jnp.float32),
            pltpu.SemaphoreType.DMA,
        ],
    )
    def gather_k(ctx_hbm, table_hbm, out_hbm, idx_v, rows_v, sem):
        cid = lax.axis_index("c")
        sid = lax.axis_index("s")

        @pl.when(jnp.logical_and(cid == 0, sid == 0))
        def _():
            pltpu.sync_copy(ctx_hbm, idx_v)
            pltpu.async_copy(table_hbm.at[idx_v], rows_v, sem).wait()
            pltpu.sync_copy(rows_v, out_hbm)

    return gather_k(ctx_pad, emb_table)


def _mlp_body(emb_ref, w1_ref, b1_ref, w2a_ref, w2b_ref, b2a_ref, b2b_ref,
              out_ref, h_ref):
    i = pl.program_id(0)

    @pl.when(i == 0)
    def _():
        acc = jnp.zeros((1, HIDDEN), jnp.float32)
        for j in range(CONTEXT):
            acc += jnp.dot(
                emb_ref[j : j + 1, :],
                w1_ref[EMB_DIM * j : EMB_DIM * (j + 1), :],
                preferred_element_type=jnp.float32,
            )
        h_ref[...] = jnp.maximum(acc + b1_ref[...], 0.0)

    h = h_ref[...]
    for s, (w2_ref, b2_ref) in enumerate(((w2a_ref, b2a_ref), (w2b_ref, b2b_ref))):
        logits = (
            jnp.dot(h, w2_ref[...], preferred_element_type=jnp.float32)
            + b2_ref[...]
        )
        col = (NS * i + s) * CHUNK + lax.broadcasted_iota(jnp.int32, (1, CHUNK), 1)
        logits = jnp.where(col < N_HEROES, logits, -1e30)
        out_ref[pl.ds(i, 1), s * CHUNK : (s + 1) * CHUNK] = logits

    @pl.when(i == K_STEPS - 1)
    def _():
        full = out_ref[...]
        m = jnp.max(full)
        s = jnp.sum(jnp.exp(full - m))
        out_ref[...] = full - (m + jnp.log(s))


def _tc_mlp(embeds, W1, b1r, W2, b2r):
    return pl.pallas_call(
        _mlp_body,
        grid=(K_STEPS,),
        in_specs=[
            pl.BlockSpec((16, EMB_DIM), lambda i: (0, 0)),
            pl.BlockSpec((CONTEXT * EMB_DIM, HIDDEN), lambda i: (0, 0)),
            pl.BlockSpec((1, HIDDEN), lambda i: (0, 0)),
            pl.BlockSpec((HIDDEN, CHUNK), lambda i: (0, NS * i)),
            pl.BlockSpec((HIDDEN, CHUNK), lambda i: (0, NS * i + 1)),
            pl.BlockSpec((1, CHUNK), lambda i: (0, NS * i)),
            pl.BlockSpec((1, CHUNK), lambda i: (0, NS * i + 1)),
        ],
        out_specs=pl.BlockSpec((K_STEPS, ROW), lambda i: (0, 0)),
        out_shape=jax.ShapeDtypeStruct((K_STEPS, ROW), jnp.float32),
        scratch_shapes=[pltpu.VMEM((1, HIDDEN), jnp.float32)],
    )(embeds, W1, b1r, W2, W2, b2r, b2r)


def kernel(context, emb_table, W1, b1, W2, b2):
    ctx_pad = jnp.zeros((16,), jnp.int32).at[:CONTEXT].set(context.astype(jnp.int32))
    embeds = _sc_gather(ctx_pad, emb_table)
    out = _tc_mlp(embeds, W1, b1.reshape(1, HIDDEN), W2, b2.reshape(1, N_HEROES))
    return out.reshape(1, PADN)[:, :N_HEROES]
